# baseline (speedup 1.0000x reference)
import functools

import jax
import jax.numpy as jnp
from jax import lax
from jax.experimental import pallas as pl
from jax.experimental.pallas import tpu as pltpu
from jax.experimental.pallas import tpu_sc as plsc

B = 8
T = 8192
D = 1024
H = 512
L = 32
NC = 256
KEEP = 128
TBLK = 512
NT = T // TBLK
CPB = TBLK // L

PS = (1, 1, 2, 2, 1, 1)
OFFS = (0, 1, 2, 4, 6, 7)
OUT_ROWS = B * KEEP * L
CH = 32


def _score_body(k_ref, v_ref, w1_ref, b1_ref, w2t_ref, out_ref):
    x = (k_ref[0] + v_ref[0]) * 0.5
    h = jnp.dot(x, w1_ref[...])
    h = jnp.maximum(h + b1_ref[...], 0.0)
    s_row = lax.dot_general(w2t_ref[...], h,
                            dimension_numbers=(((1,), (1,)), ((), ())))
    tok = lax.broadcasted_iota(jnp.int32, (TBLK, CPB), 0)
    chk = lax.broadcasted_iota(jnp.int32, (TBLK, CPB), 1)
    m2 = (tok // L == chk).astype(jnp.float32)
    c_row = jnp.dot(s_row, m2, precision=lax.Precision.HIGHEST)
    out_ref[...] = c_row.reshape(1, 1, 1, CPB)


def _scores(keys, values, W1, b1, W2, p):
    off, pb = OFFS[p], PS[p]
    return pl.pallas_call(
        _score_body,
        grid=(pb, NT),
        in_specs=[
            pl.BlockSpec((1, TBLK, D), lambda b, t: (off + b, t, 0)),
            pl.BlockSpec((1, TBLK, D), lambda b, t: (off + b, t, 0)),
            pl.BlockSpec((D, H), lambda b, t: (0, 0)),
            pl.BlockSpec((1, H), lambda b, t: (0, 0)),
            pl.BlockSpec((1, H), lambda b, t: (0, 0)),
        ],
        out_specs=pl.BlockSpec((1, 1, 1, CPB), lambda b, t: (b, t, 0, 0)),
        out_shape=jax.ShapeDtypeStruct((pb, NT, 1, CPB), jnp.float32),
    )(keys, values, W1, b1.reshape(1, H), W2.reshape(1, H)).reshape(pb, NC)


def _make_topk_body(p):
    off, pb = OFFS[p], PS[p]

    def _topk_body(cs_ref, out_ref):
        s = cs_ref[...]
        si = s[:, None, :]
        sj = s[:, :, None]
        ii = lax.broadcasted_iota(jnp.int32, (pb, NC, NC), 2)
        jj = lax.broadcasted_iota(jnp.int32, (pb, NC, NC), 1)
        gt = (sj > si).astype(jnp.float32)
        eq = ((sj == si) & (jj < ii)).astype(jnp.float32)
        cnt = jnp.sum(gt + eq, axis=1)
        keepf = (cnt < float(KEEP)).astype(jnp.float32)
        lt = (lax.broadcasted_iota(jnp.int32, (NC, NC), 0)
              < lax.broadcasted_iota(jnp.int32, (NC, NC), 1)).astype(jnp.float32)
        rank = jnp.dot(keepf, lt, precision=lax.Precision.HIGHEST)
        ranki = rank.astype(jnp.int32)
        piota = lax.broadcasted_iota(jnp.int32, (pb, KEEP, NC), 1)
        slot = ((ranki[:, None, :] == piota)
                & (keepf[:, None, :] > 0.0)).astype(jnp.int32)
        ival = lax.broadcasted_iota(jnp.int32, (pb, KEEP, NC), 2)
        chunk3 = jnp.sum(slot * ival, axis=2, keepdims=True)
        l_io = lax.broadcasted_iota(jnp.int32, (pb, KEEP, L), 2)
        b_io = lax.broadcasted_iota(jnp.int32, (pb, KEEP, L), 0)
        out_ref[...] = (b_io + off) * T + chunk3 * L + l_io

    return _topk_body


def _topk_rows(chunk_scores, p):
    return pl.pallas_call(
        _make_topk_body(p),
        out_shape=jax.ShapeDtypeStruct((PS[p], KEEP, L), jnp.int32),
    )(chunk_scores)


def _make_gather_body(p):
    rows_p = PS[p] * KEEP * L
    rpw = rows_p // 16
    nb = rpw // CH

    def _gather_body(keys_ref, vals_ref, idx_ref, outk_ref, outv_ref,
                     idxv, buf0, buf1, sem0, sem1):
        c = lax.axis_index("c")
        s = lax.axis_index("s")
        base_in = s * rpw
        base_out = OFFS[p] * KEEP * L + s * rpw
        pltpu.sync_copy(idx_ref.at[pl.ds(base_in, rpw)], idxv.at[pl.ds(0, rpw)])

        def run(table, out):
            pltpu.async_copy(table.at[idxv.at[pl.ds(0, CH)]], buf0, sem0)

            def body(i, carry):
                g0 = (2 * i) * CH
                g1 = g0 + CH
                g2 = g1 + CH
                pltpu.make_async_copy(table.at[pl.ds(0, CH)], buf0,
                                      sem0).wait()
                pltpu.async_copy(table.at[idxv.at[pl.ds(g1, CH)]], buf1, sem1)
                pltpu.sync_copy(buf0, out.at[pl.ds(base_out + g0, CH)])
                pltpu.make_async_copy(table.at[pl.ds(0, CH)], buf1,
                                      sem1).wait()

                @pl.when(i < nb // 2 - 1)
                def _():
                    pltpu.async_copy(table.at[idxv.at[pl.ds(g2, CH)]], buf0,
                                     sem0)

                pltpu.sync_copy(buf1, out.at[pl.ds(base_out + g1, CH)])
                return carry

            lax.fori_loop(0, nb // 2, body, 0)

        @pl.when(c == 0)
        def _():
            run(keys_ref, outk_ref)

        @pl.when(c == 1)
        def _():
            run(vals_ref, outv_ref)

    return _gather_body


_SCRATCH = [
    pltpu.VMEM((512,), jnp.int32),
    pltpu.VMEM((CH, D), jnp.float32),
    pltpu.VMEM((CH, D), jnp.float32),
    pltpu.SemaphoreType.DMA,
    pltpu.SemaphoreType.DMA,
]


@functools.cache
def _gather_first():
    return pl.kernel(
        _make_gather_body(0),
        mesh=plsc.VectorSubcoreMesh(core_axis_name="c", subcore_axis_name="s"),
        out_type=(jax.ShapeDtypeStruct((OUT_ROWS, D), jnp.float32),
                  jax.ShapeDtypeStruct((OUT_ROWS, D), jnp.float32)),
        scratch_types=_SCRATCH,
    )


@functools.cache
def _gather_piece(p):
    return pl.kernel(
        _make_gather_body(p),
        mesh=plsc.VectorSubcoreMesh(core_axis_name="c", subcore_axis_name="s"),
        out_type=(),
        scratch_types=_SCRATCH,
    )


def kernel(keys, values, W1, b1, W2, b2):
    del b2
    keys2d = keys.reshape(B * T, D)
    vals2d = values.reshape(B * T, D)
    rk = rv = None
    for p in range(len(PS)):
        cs = _scores(keys, values, W1, b1, W2, p)
        rows = _topk_rows(cs, p).reshape(PS[p] * KEEP * L)
        if p == 0:
            outk, outv = _gather_first()(keys2d, vals2d, rows)
            rk, rv = jax.new_ref(outk), jax.new_ref(outv)
        else:
            _gather_piece(p)(keys2d, vals2d, rows, rk, rv)
    return (rk[...].reshape(B, KEEP * L, D), rv[...].reshape(B, KEEP * L, D))


# speedup vs baseline: 1.0332x; 1.0332x over previous
import functools

import jax
import jax.numpy as jnp
from jax import lax
from jax.experimental import pallas as pl
from jax.experimental.pallas import tpu as pltpu
from jax.experimental.pallas import tpu_sc as plsc

B = 8
T = 8192
D = 1024
H = 512
L = 32
NC = 256
KEEP = 128
TBLK = 512
NT = T // TBLK
CPB = TBLK // L

PS = (1, 2, 2, 2, 1)
OFFS = (0, 1, 3, 5, 7)
OUT_ROWS = B * KEEP * L
CH = 32


def _make_score_topk_body(p):
    off, pb = OFFS[p], PS[p]

    def body(k_ref, v_ref, w1_ref, b1_ref, w2t_ref, out_ref, cs_ref):
        b = pl.program_id(0)
        t = pl.program_id(1)
        x = (k_ref[0] + v_ref[0]) * 0.5
        h = jnp.dot(x, w1_ref[...])
        h = jnp.maximum(h + b1_ref[...], 0.0)
        s_row = lax.dot_general(w2t_ref[...], h,
                                dimension_numbers=(((1,), (1,)), ((), ())))
        tok = lax.broadcasted_iota(jnp.int32, (TBLK, CPB), 0)
        chk = lax.broadcasted_iota(jnp.int32, (TBLK, CPB), 1)
        m2 = (tok // L == chk).astype(jnp.float32)
        c_row = jnp.dot(s_row, m2, precision=lax.Precision.HIGHEST)
        pr = lax.broadcasted_iota(jnp.int32, (CPB, NC), 0)
        pc = lax.broadcasted_iota(jnp.int32, (CPB, NC), 1)
        placed = jnp.dot(c_row, (pc == t * CPB + pr).astype(jnp.float32),
                         precision=lax.Precision.HIGHEST)

        @pl.when(t == 0)
        def _():
            cs_ref[pl.ds(b, 1), :] = placed

        @pl.when(t > 0)
        def _():
            cs_ref[pl.ds(b, 1), :] = cs_ref[pl.ds(b, 1), :] + placed

        @pl.when((b == pb - 1) & (t == NT - 1))
        def _():
            _topk(cs_ref[...], out_ref)

    def _topk(s, out_ref):
        si = s[:, None, :]
        sj = s[:, :, None]
        ii = lax.broadcasted_iota(jnp.int32, (pb, NC, NC), 2)
        jj = lax.broadcasted_iota(jnp.int32, (pb, NC, NC), 1)
        gt = (sj > si).astype(jnp.float32)
        eq = ((sj == si) & (jj < ii)).astype(jnp.float32)
        cnt = jnp.sum(gt + eq, axis=1)
        keepf = (cnt < float(KEEP)).astype(jnp.float32)
        lt = (lax.broadcasted_iota(jnp.int32, (NC, NC), 0)
              < lax.broadcasted_iota(jnp.int32, (NC, NC), 1)).astype(jnp.float32)
        rank = jnp.dot(keepf, lt, precision=lax.Precision.HIGHEST)
        ranki = rank.astype(jnp.int32)
        piota = lax.broadcasted_iota(jnp.int32, (pb, KEEP, NC), 1)
        slot = ((ranki[:, None, :] == piota)
                & (keepf[:, None, :] > 0.0)).astype(jnp.int32)
        ival = lax.broadcasted_iota(jnp.int32, (pb, KEEP, NC), 2)
        chunk3 = jnp.sum(slot * ival, axis=2, keepdims=True)
        l_io = lax.broadcasted_iota(jnp.int32, (pb, KEEP, L), 2)
        b_io = lax.broadcasted_iota(jnp.int32, (pb, KEEP, L), 0)
        out_ref[...] = (b_io + off) * T + chunk3 * L + l_io

    return body


def _score_topk(keys, values, W1, b1, W2, p):
    off, pb = OFFS[p], PS[p]
    return pl.pallas_call(
        _make_score_topk_body(p),
        grid=(pb, NT),
        in_specs=[
            pl.BlockSpec((1, TBLK, D), lambda b, t: (off + b, t, 0)),
            pl.BlockSpec((1, TBLK, D), lambda b, t: (off + b, t, 0)),
            pl.BlockSpec((D, H), lambda b, t: (0, 0)),
            pl.BlockSpec((1, H), lambda b, t: (0, 0)),
            pl.BlockSpec((1, H), lambda b, t: (0, 0)),
        ],
        out_specs=pl.BlockSpec((pb, KEEP, L), lambda b, t: (0, 0, 0)),
        out_shape=jax.ShapeDtypeStruct((pb, KEEP, L), jnp.int32),
        scratch_shapes=[pltpu.VMEM((pb, NC), jnp.float32)],
    )(keys, values, W1, b1.reshape(1, H), W2.reshape(1, H))


def _make_gather_body(p):
    rows_p = PS[p] * KEEP * L
    rpw = rows_p // 16
    nb = rpw // CH

    def _gather_body(keys_ref, vals_ref, idx_ref, outk_ref, outv_ref,
                     idxv, buf0, buf1, sem0, sem1):
        c = lax.axis_index("c")
        s = lax.axis_index("s")
        base_in = s * rpw
        base_out = OFFS[p] * KEEP * L + s * rpw
        pltpu.sync_copy(idx_ref.at[pl.ds(base_in, rpw)], idxv.at[pl.ds(0, rpw)])

        def run(table, out):
            pltpu.async_copy(table.at[idxv.at[pl.ds(0, CH)]], buf0, sem0)

            def body(i, carry):
                g0 = (2 * i) * CH
                g1 = g0 + CH
                g2 = g1 + CH
                pltpu.make_async_copy(table.at[pl.ds(0, CH)], buf0,
                                      sem0).wait()
                pltpu.async_copy(table.at[idxv.at[pl.ds(g1, CH)]], buf1, sem1)
                pltpu.sync_copy(buf0, out.at[pl.ds(base_out + g0, CH)])
                pltpu.make_async_copy(table.at[pl.ds(0, CH)], buf1,
                                      sem1).wait()

                @pl.when(i < nb // 2 - 1)
                def _():
                    pltpu.async_copy(table.at[idxv.at[pl.ds(g2, CH)]], buf0,
                                     sem0)

                pltpu.sync_copy(buf1, out.at[pl.ds(base_out + g1, CH)])
                return carry

            lax.fori_loop(0, nb // 2, body, 0)

        @pl.when(c == 0)
        def _():
            run(keys_ref, outk_ref)

        @pl.when(c == 1)
        def _():
            run(vals_ref, outv_ref)

    return _gather_body


_SCRATCH = [
    pltpu.VMEM((512,), jnp.int32),
    pltpu.VMEM((CH, D), jnp.float32),
    pltpu.VMEM((CH, D), jnp.float32),
    pltpu.SemaphoreType.DMA,
    pltpu.SemaphoreType.DMA,
]


@functools.cache
def _gather_first():
    return pl.kernel(
        _make_gather_body(0),
        mesh=plsc.VectorSubcoreMesh(core_axis_name="c", subcore_axis_name="s"),
        out_type=(jax.ShapeDtypeStruct((OUT_ROWS, D), jnp.float32),
                  jax.ShapeDtypeStruct((OUT_ROWS, D), jnp.float32)),
        scratch_types=_SCRATCH,
    )


@functools.cache
def _gather_piece(p):
    return pl.kernel(
        _make_gather_body(p),
        mesh=plsc.VectorSubcoreMesh(core_axis_name="c", subcore_axis_name="s"),
        out_type=(),
        scratch_types=_SCRATCH,
    )


def kernel(keys, values, W1, b1, W2, b2):
    del b2
    keys2d = keys.reshape(B * T, D)
    vals2d = values.reshape(B * T, D)
    rk = rv = None
    for p in range(len(PS)):
        rows = _score_topk(keys, values, W1, b1, W2, p).reshape(
            PS[p] * KEEP * L)
        if p == 0:
            outk, outv = _gather_first()(keys2d, vals2d, rows)
            rk, rv = jax.new_ref(outk), jax.new_ref(outv)
        else:
            _gather_piece(p)(keys2d, vals2d, rows, rk, rv)
    return (rk[...].reshape(B, KEEP * L, D), rv[...].reshape(B, KEEP * L, D))


# speedup vs baseline: 1.0999x; 1.0646x over previous
import functools

import jax
import jax.numpy as jnp
from jax import lax
from jax.experimental import pallas as pl
from jax.experimental.pallas import tpu as pltpu
from jax.experimental.pallas import tpu_sc as plsc

B = 8
T = 8192
D = 1024
H = 512
L = 32
NC = 256
KEEP = 128
TBLK = 1024
NT = T // TBLK
CPB = TBLK // L

PS = (1, 2, 2, 2, 1)
OFFS = (0, 1, 3, 5, 7)
OUT_ROWS = B * KEEP * L
CH = 32


def _make_score_topk_body(p):
    off, pb = OFFS[p], PS[p]

    def body(k_ref, v_ref, w1_ref, b1_ref, w2t_ref, out_ref, cs_ref):
        b = pl.program_id(0)
        t = pl.program_id(1)
        x = (k_ref[0] + v_ref[0]) * 0.5
        h = jnp.dot(x, w1_ref[...])
        h = jnp.maximum(h + b1_ref[...], 0.0)
        s_row = lax.dot_general(w2t_ref[...], h,
                                dimension_numbers=(((1,), (1,)), ((), ())))
        tok = lax.broadcasted_iota(jnp.int32, (TBLK, CPB), 0)
        chk = lax.broadcasted_iota(jnp.int32, (TBLK, CPB), 1)
        m2 = (tok // L == chk).astype(jnp.float32)
        c_row = jnp.dot(s_row, m2, precision=lax.Precision.HIGHEST)
        pr = lax.broadcasted_iota(jnp.int32, (CPB, NC), 0)
        pc = lax.broadcasted_iota(jnp.int32, (CPB, NC), 1)
        placed = jnp.dot(c_row, (pc == t * CPB + pr).astype(jnp.float32),
                         precision=lax.Precision.HIGHEST)

        @pl.when(t == 0)
        def _():
            cs_ref[pl.ds(b, 1), :] = placed

        @pl.when(t > 0)
        def _():
            cs_ref[pl.ds(b, 1), :] = cs_ref[pl.ds(b, 1), :] + placed

        @pl.when((b == pb - 1) & (t == NT - 1))
        def _():
            _topk(cs_ref[...], out_ref)

    def _topk(s, out_ref):
        si = s[:, None, :]
        sj = s[:, :, None]
        ii = lax.broadcasted_iota(jnp.int32, (pb, NC, NC), 2)
        jj = lax.broadcasted_iota(jnp.int32, (pb, NC, NC), 1)
        gt = (sj > si).astype(jnp.float32)
        eq = ((sj == si) & (jj < ii)).astype(jnp.float32)
        cnt = jnp.sum(gt + eq, axis=1)
        keepf = (cnt < float(KEEP)).astype(jnp.float32)
        lt = (lax.broadcasted_iota(jnp.int32, (NC, NC), 0)
              < lax.broadcasted_iota(jnp.int32, (NC, NC), 1)).astype(jnp.float32)
        rank = jnp.dot(keepf, lt, precision=lax.Precision.HIGHEST)
        ranki = rank.astype(jnp.int32)
        piota = lax.broadcasted_iota(jnp.int32, (pb, KEEP, NC), 1)
        slot = ((ranki[:, None, :] == piota)
                & (keepf[:, None, :] > 0.0)).astype(jnp.int32)
        ival = lax.broadcasted_iota(jnp.int32, (pb, KEEP, NC), 2)
        chunk3 = jnp.sum(slot * ival, axis=2, keepdims=True)
        l_io = lax.broadcasted_iota(jnp.int32, (pb, KEEP, L), 2)
        b_io = lax.broadcasted_iota(jnp.int32, (pb, KEEP, L), 0)
        out_ref[...] = (b_io + off) * T + chunk3 * L + l_io

    return body


def _score_topk(keys, values, W1, b1, W2, p):
    off, pb = OFFS[p], PS[p]
    return pl.pallas_call(
        _make_score_topk_body(p),
        grid=(pb, NT),
        in_specs=[
            pl.BlockSpec((1, TBLK, D), lambda b, t: (off + b, t, 0)),
            pl.BlockSpec((1, TBLK, D), lambda b, t: (off + b, t, 0)),
            pl.BlockSpec((D, H), lambda b, t: (0, 0)),
            pl.BlockSpec((1, H), lambda b, t: (0, 0)),
            pl.BlockSpec((1, H), lambda b, t: (0, 0)),
        ],
        out_specs=pl.BlockSpec((pb, KEEP, L), lambda b, t: (0, 0, 0)),
        out_shape=jax.ShapeDtypeStruct((pb, KEEP, L), jnp.int32),
        scratch_shapes=[pltpu.VMEM((pb, NC), jnp.float32)],
    )(keys, values, W1, b1.reshape(1, H), W2.reshape(1, H))


def _make_gather_body(p):
    rows_p = PS[p] * KEEP * L
    rpw = rows_p // 16
    nb = rpw // CH

    def _gather_body(keys_ref, vals_ref, idx_ref, outk_ref, outv_ref,
                     idxv, buf0, buf1, sem0, sem1):
        c = lax.axis_index("c")
        s = lax.axis_index("s")
        base_in = s * rpw
        base_out = OFFS[p] * KEEP * L + s * rpw
        pltpu.sync_copy(idx_ref.at[pl.ds(base_in, rpw)], idxv.at[pl.ds(0, rpw)])

        def run(table, out):
            pltpu.async_copy(table.at[idxv.at[pl.ds(0, CH)]], buf0, sem0)

            def body(i, carry):
                g0 = (2 * i) * CH
                g1 = g0 + CH
                g2 = g1 + CH
                pltpu.make_async_copy(table.at[pl.ds(0, CH)], buf0,
                                      sem0).wait()
                pltpu.async_copy(table.at[idxv.at[pl.ds(g1, CH)]], buf1, sem1)
                pltpu.sync_copy(buf0, out.at[pl.ds(base_out + g0, CH)])
                pltpu.make_async_copy(table.at[pl.ds(0, CH)], buf1,
                                      sem1).wait()

                @pl.when(i < nb // 2 - 1)
                def _():
                    pltpu.async_copy(table.at[idxv.at[pl.ds(g2, CH)]], buf0,
                                     sem0)

                pltpu.sync_copy(buf1, out.at[pl.ds(base_out + g1, CH)])
                return carry

            lax.fori_loop(0, nb // 2, body, 0)

        @pl.when(c == 0)
        def _():
            run(keys_ref, outk_ref)

        @pl.when(c == 1)
        def _():
            run(vals_ref, outv_ref)

    return _gather_body


_SCRATCH = [
    pltpu.VMEM((512,), jnp.int32),
    pltpu.VMEM((CH, D), jnp.float32),
    pltpu.VMEM((CH, D), jnp.float32),
    pltpu.SemaphoreType.DMA,
    pltpu.SemaphoreType.DMA,
]


@functools.cache
def _gather_first():
    return pl.kernel(
        _make_gather_body(0),
        mesh=plsc.VectorSubcoreMesh(core_axis_name="c", subcore_axis_name="s"),
        out_type=(jax.ShapeDtypeStruct((OUT_ROWS, D), jnp.float32),
                  jax.ShapeDtypeStruct((OUT_ROWS, D), jnp.float32)),
        scratch_types=_SCRATCH,
    )


@functools.cache
def _gather_piece(p):
    return pl.kernel(
        _make_gather_body(p),
        mesh=plsc.VectorSubcoreMesh(core_axis_name="c", subcore_axis_name="s"),
        out_type=(),
        scratch_types=_SCRATCH,
    )


def kernel(keys, values, W1, b1, W2, b2):
    del b2
    keys2d = keys.reshape(B * T, D)
    vals2d = values.reshape(B * T, D)
    rk = rv = None
    for p in range(len(PS)):
        rows = _score_topk(keys, values, W1, b1, W2, p).reshape(
            PS[p] * KEEP * L)
        if p == 0:
            outk, outv = _gather_first()(keys2d, vals2d, rows)
            rk, rv = jax.new_ref(outk), jax.new_ref(outv)
        else:
            _gather_piece(p)(keys2d, vals2d, rows, rk, rv)
    return (rk[...].reshape(B, KEEP * L, D), rv[...].reshape(B, KEEP * L, D))


# speedup vs baseline: 1.1040x; 1.0037x over previous
import functools

import jax
import jax.numpy as jnp
from jax import lax
from jax.experimental import pallas as pl
from jax.experimental.pallas import tpu as pltpu
from jax.experimental.pallas import tpu_sc as plsc

B = 8
T = 8192
D = 1024
H = 512
L = 32
NC = 256
KEEP = 128
TBLK = 2048
NT = T // TBLK
CPB = TBLK // L

PS = (1, 2, 2, 2, 1)
OFFS = (0, 1, 3, 5, 7)
OUT_ROWS = B * KEEP * L
CH = 32


def _make_score_topk_body(p):
    off, pb = OFFS[p], PS[p]

    def body(k_ref, v_ref, w1_ref, b1_ref, w2t_ref, out_ref, cs_ref):
        b = pl.program_id(0)
        t = pl.program_id(1)
        x = (k_ref[0] + v_ref[0]) * 0.5
        h = jnp.dot(x, w1_ref[...])
        h = jnp.maximum(h + b1_ref[...], 0.0)
        s_row = lax.dot_general(w2t_ref[...], h,
                                dimension_numbers=(((1,), (1,)), ((), ())))
        tok = lax.broadcasted_iota(jnp.int32, (TBLK, CPB), 0)
        chk = lax.broadcasted_iota(jnp.int32, (TBLK, CPB), 1)
        m2 = (tok // L == chk).astype(jnp.float32)
        c_row = jnp.dot(s_row, m2, precision=lax.Precision.HIGHEST)
        pr = lax.broadcasted_iota(jnp.int32, (CPB, NC), 0)
        pc = lax.broadcasted_iota(jnp.int32, (CPB, NC), 1)
        placed = jnp.dot(c_row, (pc == t * CPB + pr).astype(jnp.float32),
                         precision=lax.Precision.HIGHEST)

        @pl.when(t == 0)
        def _():
            cs_ref[pl.ds(b, 1), :] = placed

        @pl.when(t > 0)
        def _():
            cs_ref[pl.ds(b, 1), :] = cs_ref[pl.ds(b, 1), :] + placed

        @pl.when((b == pb - 1) & (t == NT - 1))
        def _():
            _topk(cs_ref[...], out_ref)

    def _topk(s, out_ref):
        si = s[:, None, :]
        sj = s[:, :, None]
        ii = lax.broadcasted_iota(jnp.int32, (pb, NC, NC), 2)
        jj = lax.broadcasted_iota(jnp.int32, (pb, NC, NC), 1)
        gt = (sj > si).astype(jnp.float32)
        eq = ((sj == si) & (jj < ii)).astype(jnp.float32)
        cnt = jnp.sum(gt + eq, axis=1)
        keepf = (cnt < float(KEEP)).astype(jnp.float32)
        lt = (lax.broadcasted_iota(jnp.int32, (NC, NC), 0)
              < lax.broadcasted_iota(jnp.int32, (NC, NC), 1)).astype(jnp.float32)
        rank = jnp.dot(keepf, lt, precision=lax.Precision.HIGHEST)
        ranki = rank.astype(jnp.int32)
        piota = lax.broadcasted_iota(jnp.int32, (pb, KEEP, NC), 1)
        slot = ((ranki[:, None, :] == piota)
                & (keepf[:, None, :] > 0.0)).astype(jnp.int32)
        ival = lax.broadcasted_iota(jnp.int32, (pb, KEEP, NC), 2)
        chunk3 = jnp.sum(slot * ival, axis=2, keepdims=True)
        l_io = lax.broadcasted_iota(jnp.int32, (pb, KEEP, L), 2)
        b_io = lax.broadcasted_iota(jnp.int32, (pb, KEEP, L), 0)
        out_ref[...] = (b_io + off) * T + chunk3 * L + l_io

    return body


def _score_topk(keys, values, W1, b1, W2, p):
    off, pb = OFFS[p], PS[p]
    return pl.pallas_call(
        _make_score_topk_body(p),
        grid=(pb, NT),
        in_specs=[
            pl.BlockSpec((1, TBLK, D), lambda b, t: (off + b, t, 0)),
            pl.BlockSpec((1, TBLK, D), lambda b, t: (off + b, t, 0)),
            pl.BlockSpec((D, H), lambda b, t: (0, 0)),
            pl.BlockSpec((1, H), lambda b, t: (0, 0)),
            pl.BlockSpec((1, H), lambda b, t: (0, 0)),
        ],
        out_specs=pl.BlockSpec((pb, KEEP, L), lambda b, t: (0, 0, 0)),
        out_shape=jax.ShapeDtypeStruct((pb, KEEP, L), jnp.int32),
        scratch_shapes=[pltpu.VMEM((pb, NC), jnp.float32)],
    )(keys, values, W1, b1.reshape(1, H), W2.reshape(1, H))


def _make_gather_body(p):
    rows_p = PS[p] * KEEP * L
    rpw = rows_p // 16
    nb = rpw // CH

    def _gather_body(keys_ref, vals_ref, idx_ref, outk_ref, outv_ref,
                     idxv, buf0, buf1, sem0, sem1):
        c = lax.axis_index("c")
        s = lax.axis_index("s")
        base_in = s * rpw
        base_out = OFFS[p] * KEEP * L + s * rpw
        pltpu.sync_copy(idx_ref.at[pl.ds(base_in, rpw)], idxv.at[pl.ds(0, rpw)])

        def run(table, out):
            pltpu.async_copy(table.at[idxv.at[pl.ds(0, CH)]], buf0, sem0)

            def body(i, carry):
                g0 = (2 * i) * CH
                g1 = g0 + CH
                g2 = g1 + CH
                pltpu.make_async_copy(table.at[pl.ds(0, CH)], buf0,
                                      sem0).wait()
                pltpu.async_copy(table.at[idxv.at[pl.ds(g1, CH)]], buf1, sem1)
                pltpu.sync_copy(buf0, out.at[pl.ds(base_out + g0, CH)])
                pltpu.make_async_copy(table.at[pl.ds(0, CH)], buf1,
                                      sem1).wait()

                @pl.when(i < nb // 2 - 1)
                def _():
                    pltpu.async_copy(table.at[idxv.at[pl.ds(g2, CH)]], buf0,
                                     sem0)

                pltpu.sync_copy(buf1, out.at[pl.ds(base_out + g1, CH)])
                return carry

            lax.fori_loop(0, nb // 2, body, 0)

        @pl.when(c == 0)
        def _():
            run(keys_ref, outk_ref)

        @pl.when(c == 1)
        def _():
            run(vals_ref, outv_ref)

    return _gather_body


_SCRATCH = [
    pltpu.VMEM((512,), jnp.int32),
    pltpu.VMEM((CH, D), jnp.float32),
    pltpu.VMEM((CH, D), jnp.float32),
    pltpu.SemaphoreType.DMA,
    pltpu.SemaphoreType.DMA,
]


@functools.cache
def _gather_first():
    return pl.kernel(
        _make_gather_body(0),
        mesh=plsc.VectorSubcoreMesh(core_axis_name="c", subcore_axis_name="s"),
        out_type=(jax.ShapeDtypeStruct((OUT_ROWS, D), jnp.float32),
                  jax.ShapeDtypeStruct((OUT_ROWS, D), jnp.float32)),
        scratch_types=_SCRATCH,
    )


@functools.cache
def _gather_piece(p):
    return pl.kernel(
        _make_gather_body(p),
        mesh=plsc.VectorSubcoreMesh(core_axis_name="c", subcore_axis_name="s"),
        out_type=(),
        scratch_types=_SCRATCH,
    )


def kernel(keys, values, W1, b1, W2, b2):
    del b2
    keys2d = keys.reshape(B * T, D)
    vals2d = values.reshape(B * T, D)
    rk = rv = None
    for p in range(len(PS)):
        rows = _score_topk(keys, values, W1, b1, W2, p).reshape(
            PS[p] * KEEP * L)
        if p == 0:
            outk, outv = _gather_first()(keys2d, vals2d, rows)
            rk, rv = jax.new_ref(outk), jax.new_ref(outv)
        else:
            _gather_piece(p)(keys2d, vals2d, rows, rk, rv)
    return (rk[...].reshape(B, KEEP * L, D), rv[...].reshape(B, KEEP * L, D))
